# Initial kernel scaffold; baseline (speedup 1.0000x reference)
import jax
import jax.numpy as jnp
from jax.experimental import pallas as pl


def kernel(random_vector, params):
    raise NotImplementedError("write your pallas kernel here")


# speedup vs baseline: 351.3719x; 351.3719x over previous
import numpy as np
import jax
import jax.numpy as jnp
from jax.experimental import pallas as pl

_BATCH = 512
_FEATURES = [64, 32, 16, 8]
_BRANCHES = [2, 8, 16]
_HC = 64
_T = 32


def _dot(a, b):
    return jnp.dot(a, b, preferred_element_type=jnp.float32)


def _level(x, C, f, fn, b, Wbr, bbr, ea, Wa1, ba1, Wa2, ba2, Was, bas, mpl):
    R = C * _T
    proj = _dot(x, Wbr) + bbr
    children = jnp.concatenate(
        [proj[:, cc * f:(cc + 1) * f] + x for cc in range(b)], axis=0)
    xa = _dot(x, Wa1[:f])
    ea_bias = _dot(ea, Wa1[2 * f:]) + ba1
    pre = jnp.concatenate([xa + ea_bias[cc:cc + 1, :] for cc in range(b)],
                          axis=0)
    pre = pre + _dot(children, Wa1[f:2 * f])
    m = _dot(jax.nn.relu(pre), Wa2) + ba2
    x = jax.nn.relu(_dot(children, Was) + bas + m)
    for (Wm, bm, Wu, bu) in mpl:
        A = _dot(x, Wm[:fn])
        B = _dot(x, Wm[fn:]) + bm
        A3 = A.reshape(b, R, _HC)
        B3 = B.reshape(b, R, _HC)
        acc = -jax.nn.relu(A3 + B3)
        for s in range(b):
            acc = acc + jax.nn.relu(A3[s][None, :, :] + B3)
        aggc = acc.reshape(b * R, _HC)
        x = jax.nn.relu(_dot(x, Wu[:fn]) + _dot(aggc, Wu[fn:]) + bu)
    return x


def _body(*refs):
    rv_ref = refs[0]
    out_ref = refs[-1]
    pr = refs[1:-1]
    x = rv_ref[...]
    idx = 0
    C = 1
    for l in range(3):
        f, fn, b = _FEATURES[l], _FEATURES[l + 1], _BRANCHES[l]
        vals = [r[...] for r in pr[idx:idx + 17]]
        idx += 17
        (Wbr, bbr, ea, Wa1, ba1, Wa2, ba2, Was, bas) = vals[:9]
        mpl = [tuple(vals[9 + 4 * t:9 + 4 * (t + 1)]) for t in range(2)]
        x = _level(x, C, f, fn, b, Wbr, bbr, ea, Wa1, ba1, Wa2, ba2,
                   Was, bas, mpl)
        C *= b
    out_ref[...] = x.reshape(C, _T, _FEATURES[-1])


def _const_spec(shape):
    nd = len(shape)
    return pl.BlockSpec(shape, lambda i, _nd=nd: (0,) * _nd)


_INV = np.empty((128 * 2,), dtype=np.int32)
for _p in range(256):
    _c1 = _p // 128
    _cc1 = (_p // 16) % 8
    _cc2 = _p % 16
    _INV[_p] = _cc2 * 16 + _cc1 * 2 + _c1


def kernel(random_vector, params):
    flat = []
    for l in range(3):
        p = params['lvl%d' % l]
        flat += [p['Wbr'], p['bbr'].reshape(1, -1), p['ea'], p['Wa1'],
                 p['ba1'].reshape(1, -1), p['Wa2'], p['ba2'].reshape(1, -1),
                 p['Was'], p['bas'].reshape(1, -1)]
        for t in range(2):
            m = p['mpl'][t]
            flat += [m['Wm'], m['bm'].reshape(1, -1),
                     m['Wu'], m['bu'].reshape(1, -1)]
    grid = _BATCH // _T
    out = pl.pallas_call(
        _body,
        grid=(grid,),
        in_specs=[pl.BlockSpec((_T, _FEATURES[0]), lambda i: (i, 0))] +
                 [_const_spec(a.shape) for a in flat],
        out_specs=pl.BlockSpec((256, _T, _FEATURES[-1]), lambda i: (0, i, 0)),
        out_shape=jax.ShapeDtypeStruct((256, _BATCH, _FEATURES[-1]),
                                       jnp.float32),
    )(random_vector, *flat)
    return out[_INV, :, :].transpose(1, 0, 2)


# speedup vs baseline: 673.0215x; 1.9154x over previous
import numpy as np
import jax
import jax.numpy as jnp
from jax.experimental import pallas as pl

_BATCH = 512
_FEATURES = [64, 32, 16, 8]
_BRANCHES = [2, 8, 16]
_HC = 64
_T = 64
_G = _BATCH // _T


def _dot(a, b):
    return jnp.dot(a, b, preferred_element_type=jnp.float32)


def _level_t(xT, C, f, fn, b, WbrTa, eaT, Wa1Ta, Wa2T, WasT, bias2,
             mpl):
    N = C * _T
    BN = b * N
    ones1 = jnp.ones((1, N), jnp.float32)
    onesB = jnp.ones((1, BN), jnp.float32)
    proj = _dot(WbrTa, jnp.concatenate([xT, ones1], axis=0))
    children = jnp.concatenate(
        [proj[cc * f:(cc + 1) * f, :] + xT for cc in range(b)], axis=1)
    src = jnp.concatenate([xT] * b, axis=1)
    eaRows = jnp.concatenate(
        [jnp.broadcast_to(eaT[:, cc:cc + 1], (eaT.shape[0], N))
         for cc in range(b)], axis=1)
    m_in = jnp.concatenate([src, children, eaRows, onesB], axis=0)
    h = jax.nn.relu(_dot(Wa1Ta, m_in))
    x = jax.nn.relu(_dot(WasT, children) + _dot(Wa2T, h) + bias2)
    for (Wm1T, Wm2Ta, Wu1T, Wu2T, buC) in mpl:
        A = _dot(Wm1T, x)
        B = _dot(Wm2Ta, jnp.concatenate([x, onesB], axis=0))
        aggs = []
        for d in range(b):
            Bd = B[:, d * N:(d + 1) * N]
            a = None
            for s in range(b):
                if s == d:
                    continue
                term = jax.nn.relu(A[:, s * N:(s + 1) * N] + Bd)
                a = term if a is None else a + term
            aggs.append(a)
        agg = jnp.concatenate(aggs, axis=1)
        x = jax.nn.relu(_dot(Wu1T, x) + _dot(Wu2T, agg) + buC)
    return x


def _body(*refs):
    rv_ref = refs[0]
    out_ref = refs[-1]
    pr = refs[1:-1]
    x = rv_ref[0]
    idx = 0
    C = 1
    for l in range(3):
        f, fn, b = _FEATURES[l], _FEATURES[l + 1], _BRANCHES[l]
        vals = [r[...] for r in pr[idx:idx + 16]]
        idx += 16
        (WbrTa, eaT, Wa1Ta, Wa2T, WasT, bias2) = vals[:6]
        mpl = [tuple(vals[6 + 5 * t:6 + 5 * (t + 1)]) for t in range(2)]
        x = _level_t(x, C, f, fn, b, WbrTa, eaT, Wa1Ta, Wa2T, WasT,
                     bias2, mpl)
        C *= b
    out_ref[...] = x


def _const_spec(shape):
    nd = len(shape)
    return pl.BlockSpec(shape, lambda i, _nd=nd: (0,) * _nd)


_INV = np.empty((256,), dtype=np.int32)
for _p in range(256):
    _c1 = _p // 128
    _cc1 = (_p // 16) % 8
    _cc2 = _p % 16
    _INV[_p] = _cc2 * 16 + _cc1 * 2 + _c1


def _flatten_params(params):
    flat = []
    for l in range(3):
        p = params['lvl%d' % l]
        fn = _FEATURES[l + 1]
        flat.append(jnp.concatenate(
            [p['Wbr'].T, p['bbr'][:, None]], axis=1))
        flat.append(p['ea'].T)
        flat.append(jnp.concatenate(
            [p['Wa1'].T, p['ba1'][:, None]], axis=1))
        flat.append(p['Wa2'].T)
        flat.append(p['Was'].T)
        flat.append((p['bas'] + p['ba2'])[:, None])
        for t in range(2):
            m = p['mpl'][t]
            flat.append(m['Wm'][:fn].T)
            flat.append(jnp.concatenate(
                [m['Wm'][fn:].T, m['bm'][:, None]], axis=1))
            flat.append(m['Wu'][:fn].T)
            flat.append(m['Wu'][fn:].T)
            flat.append(m['bu'][:, None])
    return flat


def kernel(random_vector, params):
    flat = _flatten_params(params)
    rvT = random_vector.T.reshape(_FEATURES[0], _G, _T)
    rvT = rvT.transpose(1, 0, 2)
    lanes_out = 256 * _T
    out = pl.pallas_call(
        _body,
        grid=(_G,),
        in_specs=[pl.BlockSpec((1, _FEATURES[0], _T), lambda i: (i, 0, 0))] +
                 [_const_spec(a.shape) for a in flat],
        out_specs=pl.BlockSpec((_FEATURES[-1], lanes_out),
                               lambda i: (0, i)),
        out_shape=jax.ShapeDtypeStruct((_FEATURES[-1], _G * lanes_out),
                                       jnp.float32),
    )(rvT, *flat)
    o = out.reshape(_FEATURES[-1], _G, 256, _T)[:, :, _INV, :]
    return o.transpose(1, 3, 2, 0).reshape(_BATCH, 256, _FEATURES[-1])


# speedup vs baseline: 685.0850x; 1.0179x over previous
import numpy as np
import jax
import jax.numpy as jnp
from jax.experimental import pallas as pl

_BATCH = 512
_FEATURES = [64, 32, 16, 8]
_BRANCHES = [2, 8, 16]
_HC = 64
_T = 64
_G = _BATCH // _T


def _dot(a, b):
    return jnp.dot(a, b, preferred_element_type=jnp.float32)


def _level_t(xT, C, f, fn, b, WbrTa, eaT, Wa1Ta, Wa2T, WasT, bias2,
             mpl):
    N = C * _T
    BN = b * N
    ones1 = jnp.ones((1, N), jnp.float32)
    onesB = jnp.ones((1, BN), jnp.float32)
    proj = _dot(WbrTa, jnp.concatenate([xT, ones1], axis=0))
    children = jnp.concatenate(
        [proj[cc * f:(cc + 1) * f, :] + xT for cc in range(b)], axis=1)
    src = jnp.concatenate([xT] * b, axis=1)
    eaRows = jnp.concatenate(
        [jnp.broadcast_to(eaT[:, cc:cc + 1], (eaT.shape[0], N))
         for cc in range(b)], axis=1)
    m_in = jnp.concatenate([src, children, eaRows, onesB], axis=0)
    h = jax.nn.relu(_dot(Wa1Ta, m_in))
    x = jax.nn.relu(_dot(WasT, children) + _dot(Wa2T, h) + bias2)
    for (Wm1T, Wm2Ta, Wu1T, Wu2T, buC) in mpl:
        A = _dot(Wm1T, x)
        B = _dot(Wm2Ta, jnp.concatenate([x, onesB], axis=0))
        aggs = []
        for d in range(b):
            Bd = B[:, d * N:(d + 1) * N]
            a = None
            for s in range(b):
                if s == d:
                    continue
                term = jax.nn.relu(A[:, s * N:(s + 1) * N] + Bd)
                a = term if a is None else a + term
            aggs.append(a)
        agg = jnp.concatenate(aggs, axis=1)
        x = jax.nn.relu(_dot(Wu1T, x) + _dot(Wu2T, agg) + buC)
    return x


def _body(*refs):
    rv_ref = refs[0]
    out_ref = refs[-1]
    pr = refs[1:-1]
    x = rv_ref[0]
    idx = 0
    C = 1
    for l in range(3):
        f, fn, b = _FEATURES[l], _FEATURES[l + 1], _BRANCHES[l]
        vals = [r[...] for r in pr[idx:idx + 16]]
        idx += 16
        (WbrTa, eaT, Wa1Ta, Wa2T, WasT, bias2) = vals[:6]
        mpl = [tuple(vals[6 + 5 * t:6 + 5 * (t + 1)]) for t in range(2)]
        x = _level_t(x, C, f, fn, b, WbrTa, eaT, Wa1Ta, Wa2T, WasT,
                     bias2, mpl)
        C *= b
    out_ref[...] = x


def _const_spec(shape):
    nd = len(shape)
    return pl.BlockSpec(shape, lambda i, _nd=nd: (0,) * _nd)


_INV = np.empty((256,), dtype=np.int32)
for _p in range(256):
    _c1 = _p // 128
    _cc1 = (_p // 16) % 8
    _cc2 = _p % 16
    _INV[_p] = _cc2 * 16 + _cc1 * 2 + _c1


def _flatten_params(params):
    flat = []
    for l in range(3):
        p = params['lvl%d' % l]
        fn = _FEATURES[l + 1]
        flat.append(jnp.concatenate(
            [p['Wbr'].T, p['bbr'][:, None]], axis=1))
        flat.append(p['ea'].T)
        flat.append(jnp.concatenate(
            [p['Wa1'].T, p['ba1'][:, None]], axis=1))
        flat.append(p['Wa2'].T)
        flat.append(p['Was'].T)
        flat.append((p['bas'] + p['ba2'])[:, None])
        for t in range(2):
            m = p['mpl'][t]
            flat.append(m['Wm'][:fn].T)
            flat.append(jnp.concatenate(
                [m['Wm'][fn:].T, m['bm'][:, None]], axis=1))
            flat.append(m['Wu'][:fn].T)
            flat.append(m['Wu'][fn:].T)
            flat.append(m['bu'][:, None])
    return flat


def kernel(random_vector, params):
    flat = _flatten_params(params)
    rvT = random_vector.T.reshape(_FEATURES[0], _G, _T)
    rvT = rvT.transpose(1, 0, 2)
    lanes_out = 256 * _T
    out = pl.pallas_call(
        _body,
        grid=(_G,),
        in_specs=[pl.BlockSpec((1, _FEATURES[0], _T), lambda i: (i, 0, 0))] +
                 [_const_spec(a.shape) for a in flat],
        out_specs=pl.BlockSpec((_FEATURES[-1], lanes_out),
                               lambda i: (0, i)),
        out_shape=jax.ShapeDtypeStruct((_FEATURES[-1], _G * lanes_out),
                                       jnp.float32),
    )(rvT, *flat)
    o = out.reshape(_FEATURES[-1], _G, 16, 8, 2, _T)
    return o.transpose(1, 5, 4, 3, 2, 0).reshape(_BATCH, 256, _FEATURES[-1])


# speedup vs baseline: 910.3618x; 1.3288x over previous
import jax
import jax.numpy as jnp
from jax.experimental import pallas as pl

_BATCH = 512
_FEATURES = [64, 32, 16, 8]
_BRANCHES = [2, 8, 16]
_HC = 64
_T = 128
_G = _BATCH // _T


def _dott(w, x):
    return jax.lax.dot_general(w, x, (((0,), (0,)), ((), ())),
                               preferred_element_type=jnp.float32)


def _level_t(xT, C, f, fn, b, Wbr, bbr, ea, Wa1, ba1, Wa2, ba2, Was, bas,
             mpl):
    N = C * _T
    BN = b * N
    ones1 = jnp.ones((1, N), jnp.float32)
    onesB = jnp.ones((1, BN), jnp.float32)
    proj = _dott(jnp.concatenate([Wbr, bbr], axis=0),
                 jnp.concatenate([xT, ones1], axis=0))
    children = jnp.concatenate(
        [proj[cc * f:(cc + 1) * f, :] + xT for cc in range(b)], axis=1)
    src = jnp.concatenate([xT] * b, axis=1)
    eaRows = jnp.concatenate(
        [jnp.broadcast_to(ea[cc][:, None], (ea.shape[1], N))
         for cc in range(b)], axis=1)
    m_in = jnp.concatenate([src, children, eaRows, onesB], axis=0)
    h = jax.nn.relu(_dott(jnp.concatenate([Wa1, ba1], axis=0), m_in))
    bias2 = jnp.swapaxes(bas + ba2, 0, 1)
    x = jax.nn.relu(_dott(Was, children) + _dott(Wa2, h) + bias2)
    for (Wm, bm, Wu, bu) in mpl:
        A = _dott(Wm[:fn], x)
        B = _dott(jnp.concatenate([Wm[fn:], bm], axis=0),
                  jnp.concatenate([x, onesB], axis=0))
        aggs = []
        for d in range(b):
            Bd = B[:, d * N:(d + 1) * N]
            a = None
            for s in range(b):
                if s == d:
                    continue
                term = jax.nn.relu(A[:, s * N:(s + 1) * N] + Bd)
                a = term if a is None else a + term
            aggs.append(a)
        agg = jnp.concatenate(aggs, axis=1)
        buC = jnp.swapaxes(bu, 0, 1)
        x = jax.nn.relu(_dott(Wu[:fn], x) + _dott(Wu[fn:], agg) + buC)
    return x


def _body(*refs):
    rv_ref = refs[0]
    out_ref = refs[-1]
    pr = refs[1:-1]
    x = jnp.swapaxes(rv_ref[...], 0, 1)
    idx = 0
    C = 1
    for l in range(3):
        f, fn, b = _FEATURES[l], _FEATURES[l + 1], _BRANCHES[l]
        vals = [r[...] for r in pr[idx:idx + 17]]
        idx += 17
        (Wbr, bbr, ea, Wa1, ba1, Wa2, ba2, Was, bas) = vals[:9]
        mpl = [tuple(vals[9 + 4 * t:9 + 4 * (t + 1)]) for t in range(2)]
        x = _level_t(x, C, f, fn, b, Wbr, bbr, ea, Wa1, ba1, Wa2, ba2,
                     Was, bas, mpl)
        C *= b
    out_ref[...] = x


def _const_spec(shape):
    nd = len(shape)
    return pl.BlockSpec(shape, lambda i, _nd=nd: (0,) * _nd)


def kernel(random_vector, params):
    flat = []
    for l in range(3):
        p = params['lvl%d' % l]
        flat += [p['Wbr'], p['bbr'][None, :], p['ea'], p['Wa1'],
                 p['ba1'][None, :], p['Wa2'], p['ba2'][None, :],
                 p['Was'], p['bas'][None, :]]
        for t in range(2):
            m = p['mpl'][t]
            flat += [m['Wm'], m['bm'][None, :], m['Wu'], m['bu'][None, :]]
    lanes_out = 256 * _T
    out = pl.pallas_call(
        _body,
        grid=(_G,),
        in_specs=[pl.BlockSpec((_T, _FEATURES[0]), lambda i: (i, 0))] +
                 [_const_spec(a.shape) for a in flat],
        out_specs=pl.BlockSpec((_FEATURES[-1], lanes_out),
                               lambda i: (0, i)),
        out_shape=jax.ShapeDtypeStruct((_FEATURES[-1], _G * lanes_out),
                                       jnp.float32),
    )(random_vector, *flat)
    o = out.reshape(_FEATURES[-1], _G, 16, 8, 2, _T)
    return o.transpose(1, 5, 4, 3, 2, 0).reshape(_BATCH, 256, _FEATURES[-1])
